# Initial kernel scaffold; baseline (speedup 1.0000x reference)
import jax
import jax.numpy as jnp
from jax.experimental import pallas as pl


def kernel(x, W1, b1, W2, b2):
    raise NotImplementedError("write your pallas kernel here")


# speedup vs baseline: 1.3450x; 1.3450x over previous
import functools

import jax
import jax.numpy as jnp
from jax.experimental import pallas as pl
from jax.experimental.pallas import tpu as pltpu

NUM_SHARED = 8
TOP_K = 8


def _router_body(x_ref, w1_ref, b1_ref, w2_ref, b2_ref,
                 weights_ref, vals_ref, idx_ref, acc_ref,
                 *, n1_tiles: int, num_experts: int):
    n1 = pl.program_id(1)

    h = jnp.dot(x_ref[...], w1_ref[...],
                preferred_element_type=jnp.float32,
                precision=jax.lax.Precision.DEFAULT)
    h = h + b1_ref[...]
    h = 0.5 * h * (1.0 + jax.lax.erf(h * 0.7071067811865476))
    part = jnp.dot(h, w2_ref[...],
                   preferred_element_type=jnp.float32,
                   precision=jax.lax.Precision.DEFAULT)

    @pl.when(n1 == 0)
    def _init():
        acc_ref[...] = part

    @pl.when(n1 != 0)
    def _accum():
        acc_ref[...] += part

    @pl.when(n1 == n1_tiles - 1)
    def _finish():
        logits = acc_ref[...] + b2_ref[...]
        m = jnp.max(logits, axis=-1, keepdims=True)
        e = jnp.exp(logits - m)
        w = e / jnp.sum(e, axis=-1, keepdims=True)
        weights_ref[...] = w

        lane = jax.lax.broadcasted_iota(jnp.int32, w.shape, 1)
        work = jnp.where(lane >= NUM_SHARED, w, -1.0)
        vals = []
        idxs = []
        for _ in range(TOP_K):
            cur = jnp.max(work, axis=-1, keepdims=True)
            cand = jnp.where(work == cur, lane, num_experts)
            sel = jnp.min(cand, axis=-1, keepdims=True)
            vals.append(cur)
            idxs.append(sel - NUM_SHARED)
            work = jnp.where(lane == sel, -1.0, work)
        vals_ref[...] = jnp.concatenate(vals, axis=-1)
        idx_ref[...] = jnp.concatenate(idxs, axis=-1)


def kernel(x, W1, b1, W2, b2):
    n_tokens, input_dim = x.shape
    hidden = W1.shape[1]
    num_experts = W2.shape[1]

    bm = min(512, n_tokens)
    bn1 = min(512, hidden)
    m_tiles = n_tokens // bm
    n1_tiles = hidden // bn1

    b1r = b1.reshape(1, hidden)
    b2r = b2.reshape(1, num_experts)

    body = functools.partial(_router_body, n1_tiles=n1_tiles,
                             num_experts=num_experts)

    weights, vals, idx = pl.pallas_call(
        body,
        grid=(m_tiles, n1_tiles),
        in_specs=[
            pl.BlockSpec((bm, input_dim), lambda i, j: (i, 0)),
            pl.BlockSpec((input_dim, bn1), lambda i, j: (0, j)),
            pl.BlockSpec((1, bn1), lambda i, j: (0, j)),
            pl.BlockSpec((bn1, num_experts), lambda i, j: (j, 0)),
            pl.BlockSpec((1, num_experts), lambda i, j: (0, 0)),
        ],
        out_specs=[
            pl.BlockSpec((bm, num_experts), lambda i, j: (i, 0)),
            pl.BlockSpec((bm, TOP_K), lambda i, j: (i, 0)),
            pl.BlockSpec((bm, TOP_K), lambda i, j: (i, 0)),
        ],
        out_shape=[
            jax.ShapeDtypeStruct((n_tokens, num_experts), jnp.float32),
            jax.ShapeDtypeStruct((n_tokens, TOP_K), jnp.float32),
            jax.ShapeDtypeStruct((n_tokens, TOP_K), jnp.int32),
        ],
        scratch_shapes=[pltpu.VMEM((bm, num_experts), jnp.float32)],
        compiler_params=pltpu.CompilerParams(
            dimension_semantics=("parallel", "arbitrary"),
        ),
    )(x, W1, b1r, W2, b2r)

    global_weights = weights[:, :NUM_SHARED]
    return (global_weights, vals, idx, weights)


# speedup vs baseline: 1.4898x; 1.1077x over previous
import functools

import jax
import jax.numpy as jnp
from jax.experimental import pallas as pl
from jax.experimental.pallas import tpu as pltpu

NUM_SHARED = 8
TOP_K = 8


def _router_body(x_ref, w1_ref, b1_ref, w2_ref, b2_ref,
                 weights_ref, vals_ref, idx_ref, acc_ref,
                 *, n1_tiles: int, num_experts: int):
    n1 = pl.program_id(1)

    h = jnp.dot(x_ref[...], w1_ref[...],
                preferred_element_type=jnp.float32,
                precision=jax.lax.Precision.DEFAULT)
    h = h + b1_ref[...]
    h = 0.5 * h * (1.0 + jax.lax.erf(h * 0.7071067811865476))
    part = jnp.dot(h, w2_ref[...],
                   preferred_element_type=jnp.float32,
                   precision=jax.lax.Precision.DEFAULT)

    @pl.when(n1 == 0)
    def _init():
        acc_ref[...] = part

    @pl.when(n1 != 0)
    def _accum():
        acc_ref[...] += part

    @pl.when(n1 == n1_tiles - 1)
    def _finish():
        logits = acc_ref[...] + b2_ref[...]
        m = jnp.max(logits, axis=-1, keepdims=True)
        e = jnp.exp(logits - m)
        w = e / jnp.sum(e, axis=-1, keepdims=True)
        weights_ref[...] = w

        lane = jax.lax.broadcasted_iota(jnp.int32, w.shape, 1)
        work = jnp.where(lane >= NUM_SHARED, w, -1.0)
        vals = []
        idxs = []
        for _ in range(TOP_K):
            cur = jnp.max(work, axis=-1, keepdims=True)
            cand = jnp.where(work == cur, lane, num_experts)
            sel = jnp.min(cand, axis=-1, keepdims=True)
            vals.append(cur)
            idxs.append(sel - NUM_SHARED)
            work = jnp.where(lane == sel, -1.0, work)
        vals_ref[...] = jnp.concatenate(vals, axis=-1)
        idx_ref[...] = jnp.concatenate(idxs, axis=-1)


def kernel(x, W1, b1, W2, b2):
    n_tokens, input_dim = x.shape
    hidden = W1.shape[1]
    num_experts = W2.shape[1]

    bm = min(512, n_tokens)
    bn1 = min(1024, hidden)
    m_tiles = n_tokens // bm
    n1_tiles = hidden // bn1

    b1r = b1.reshape(1, hidden)
    b2r = b2.reshape(1, num_experts)

    body = functools.partial(_router_body, n1_tiles=n1_tiles,
                             num_experts=num_experts)

    weights, vals, idx = pl.pallas_call(
        body,
        grid=(m_tiles, n1_tiles),
        in_specs=[
            pl.BlockSpec((bm, input_dim), lambda i, j: (i, 0)),
            pl.BlockSpec((input_dim, bn1), lambda i, j: (0, j)),
            pl.BlockSpec((1, bn1), lambda i, j: (0, j)),
            pl.BlockSpec((bn1, num_experts), lambda i, j: (j, 0)),
            pl.BlockSpec((1, num_experts), lambda i, j: (0, 0)),
        ],
        out_specs=[
            pl.BlockSpec((bm, num_experts), lambda i, j: (i, 0)),
            pl.BlockSpec((bm, TOP_K), lambda i, j: (i, 0)),
            pl.BlockSpec((bm, TOP_K), lambda i, j: (i, 0)),
        ],
        out_shape=[
            jax.ShapeDtypeStruct((n_tokens, num_experts), jnp.float32),
            jax.ShapeDtypeStruct((n_tokens, TOP_K), jnp.float32),
            jax.ShapeDtypeStruct((n_tokens, TOP_K), jnp.int32),
        ],
        scratch_shapes=[pltpu.VMEM((bm, num_experts), jnp.float32)],
        compiler_params=pltpu.CompilerParams(
            dimension_semantics=("parallel", "arbitrary"),
        ),
    )(x, W1, b1r, W2, b2r)

    global_weights = weights[:, :NUM_SHARED]
    return (global_weights, vals, idx, weights)


# speedup vs baseline: 1.5002x; 1.0070x over previous
import functools

import jax
import jax.numpy as jnp
from jax.experimental import pallas as pl
from jax.experimental.pallas import tpu as pltpu

NUM_SHARED = 8
TOP_K = 8


def _router_body(x_ref, w1_ref, b1_ref, w2_ref, b2_ref,
                 weights_ref, vals_ref, idx_ref, acc_ref,
                 *, n1_tiles: int, num_experts: int):
    n1 = pl.program_id(1)

    h = jnp.dot(x_ref[...], w1_ref[...],
                preferred_element_type=jnp.float32,
                precision=jax.lax.Precision.DEFAULT)
    h = h + b1_ref[...]
    h = 0.5 * h * (1.0 + jax.lax.erf(h * 0.7071067811865476))
    part = jnp.dot(h, w2_ref[...],
                   preferred_element_type=jnp.float32,
                   precision=jax.lax.Precision.DEFAULT)

    @pl.when(n1 == 0)
    def _init():
        acc_ref[...] = part

    @pl.when(n1 != 0)
    def _accum():
        acc_ref[...] += part

    @pl.when(n1 == n1_tiles - 1)
    def _finish():
        logits = acc_ref[...] + b2_ref[...]
        m = jnp.max(logits, axis=-1, keepdims=True)
        e = jnp.exp(logits - m)
        w = e / jnp.sum(e, axis=-1, keepdims=True)
        weights_ref[...] = w

        lane = jax.lax.broadcasted_iota(jnp.int32, w.shape, 1)
        work = jnp.where(lane >= NUM_SHARED, w, -1.0)
        vals = []
        idxs = []
        for _ in range(TOP_K):
            cur = jnp.max(work, axis=-1, keepdims=True)
            cand = jnp.where(work == cur, lane, num_experts)
            sel = jnp.min(cand, axis=-1, keepdims=True)
            vals.append(cur)
            idxs.append(sel - NUM_SHARED)
            work = jnp.where(lane == sel, -1.0, work)
        vals_ref[...] = jnp.concatenate(vals, axis=-1)
        idx_ref[...] = jnp.concatenate(idxs, axis=-1)


def kernel(x, W1, b1, W2, b2):
    n_tokens, input_dim = x.shape
    hidden = W1.shape[1]
    num_experts = W2.shape[1]

    bm = min(1024, n_tokens)
    bn1 = min(256, hidden)
    m_tiles = n_tokens // bm
    n1_tiles = hidden // bn1

    b1r = b1.reshape(1, hidden)
    b2r = b2.reshape(1, num_experts)

    body = functools.partial(_router_body, n1_tiles=n1_tiles,
                             num_experts=num_experts)

    weights, vals, idx = pl.pallas_call(
        body,
        grid=(m_tiles, n1_tiles),
        in_specs=[
            pl.BlockSpec((bm, input_dim), lambda i, j: (i, 0)),
            pl.BlockSpec((input_dim, bn1), lambda i, j: (0, j)),
            pl.BlockSpec((1, bn1), lambda i, j: (0, j)),
            pl.BlockSpec((bn1, num_experts), lambda i, j: (j, 0)),
            pl.BlockSpec((1, num_experts), lambda i, j: (0, 0)),
        ],
        out_specs=[
            pl.BlockSpec((bm, num_experts), lambda i, j: (i, 0)),
            pl.BlockSpec((bm, TOP_K), lambda i, j: (i, 0)),
            pl.BlockSpec((bm, TOP_K), lambda i, j: (i, 0)),
        ],
        out_shape=[
            jax.ShapeDtypeStruct((n_tokens, num_experts), jnp.float32),
            jax.ShapeDtypeStruct((n_tokens, TOP_K), jnp.float32),
            jax.ShapeDtypeStruct((n_tokens, TOP_K), jnp.int32),
        ],
        scratch_shapes=[pltpu.VMEM((bm, num_experts), jnp.float32)],
        compiler_params=pltpu.CompilerParams(
            dimension_semantics=("parallel", "arbitrary"),
        ),
    )(x, W1, b1r, W2, b2r)

    global_weights = weights[:, :NUM_SHARED]
    return (global_weights, vals, idx, weights)
